# baseline (speedup 1.0000x reference)
import jax
import jax.numpy as jnp
from jax.experimental import pallas as pl
from jax.experimental.pallas import tpu as pltpu

B = 4
S = 2048
IN_F = 1024
OUT_F = 1024
NUM_BANKS = 16
NS = 4
SB = S // NS


def _body(sel_ref, bw_ref, x0_ref, x1_ref, x2_ref, x3_ref, w0_ref, w1_ref,
          bias_ref, out_ref):
    b = pl.program_id(0)
    bw0 = bw_ref[b, 0]
    bw1 = bw_ref[b, 1]
    w_eff = (bw0 * w0_ref[0] + bw1 * w1_ref[0]).astype(jnp.bfloat16)
    s0 = sel_ref[b, 0]
    s1 = sel_ref[b, 1]
    b_eff = (bw0 * bias_ref[s0, :] + bw1 * bias_ref[s1, :])[None, :]
    for i, xr in enumerate((x0_ref, x1_ref, x2_ref, x3_ref)):
        out_ref[0, i * SB:(i + 1) * SB] = jnp.dot(
            xr[0].astype(jnp.bfloat16), w_eff,
            preferred_element_type=jnp.float32) + b_eff


def kernel(tensor, bank_weights, bank_selections, W, bias):
    grid_spec = pltpu.PrefetchScalarGridSpec(
        num_scalar_prefetch=2,
        grid=(B,),
        in_specs=[
            pl.BlockSpec((1, SB, IN_F), lambda b, sel, bw: (b, 0, 0)),
            pl.BlockSpec((1, SB, IN_F), lambda b, sel, bw: (b, 1, 0)),
            pl.BlockSpec((1, SB, IN_F), lambda b, sel, bw: (b, 2, 0)),
            pl.BlockSpec((1, SB, IN_F), lambda b, sel, bw: (b, 3, 0)),
            pl.BlockSpec((1, IN_F, OUT_F), lambda b, sel, bw: (sel[b, 0], 0, 0)),
            pl.BlockSpec((1, IN_F, OUT_F), lambda b, sel, bw: (sel[b, 1], 0, 0)),
            pl.BlockSpec((NUM_BANKS, OUT_F), lambda b, sel, bw: (0, 0)),
        ],
        out_specs=pl.BlockSpec((1, S, OUT_F), lambda b, sel, bw: (b, 0, 0)),
    )
    return pl.pallas_call(
        _body,
        grid_spec=grid_spec,
        out_shape=jax.ShapeDtypeStruct((B, S, OUT_F), jnp.float32),
    )(bank_selections, bank_weights, tensor, tensor, tensor, tensor, W, W, bias)


# speedup vs baseline: 1.0039x; 1.0039x over previous
import jax
import jax.numpy as jnp
from jax.experimental import pallas as pl
from jax.experimental.pallas import tpu as pltpu

B = 4
S = 2048
IN_F = 1024
OUT_F = 1024
NUM_BANKS = 16
SB = S // 2
JB = OUT_F // 2


def _body(sel_ref, bw_ref, xl_ref, xh_ref, w0a_ref, w0b_ref, w1a_ref, w1b_ref,
          bias_ref, out_ref):
    b = pl.program_id(0)
    bw0 = bw_ref[b, 0]
    bw1 = bw_ref[b, 1]
    s0 = sel_ref[b, 0]
    s1 = sel_ref[b, 1]
    b_eff = (bw0 * bias_ref[s0, :] + bw1 * bias_ref[s1, :])[None, :]
    wa = (bw0 * w0a_ref[0] + bw1 * w1a_ref[0]).astype(jnp.bfloat16)
    wb = (bw0 * w0b_ref[0] + bw1 * w1b_ref[0]).astype(jnp.bfloat16)
    xl = xl_ref[0].astype(jnp.bfloat16)
    xh = xh_ref[0].astype(jnp.bfloat16)
    out_ref[0, :SB, :JB] = jnp.dot(xl, wa, preferred_element_type=jnp.float32) + b_eff[:, :JB]
    out_ref[0, :SB, JB:] = jnp.dot(xl, wb, preferred_element_type=jnp.float32) + b_eff[:, JB:]
    out_ref[0, SB:, :JB] = jnp.dot(xh, wa, preferred_element_type=jnp.float32) + b_eff[:, :JB]
    out_ref[0, SB:, JB:] = jnp.dot(xh, wb, preferred_element_type=jnp.float32) + b_eff[:, JB:]


def kernel(tensor, bank_weights, bank_selections, W, bias):
    grid_spec = pltpu.PrefetchScalarGridSpec(
        num_scalar_prefetch=2,
        grid=(B,),
        in_specs=[
            pl.BlockSpec((1, SB, IN_F), lambda b, sel, bw: (b, 0, 0)),
            pl.BlockSpec((1, SB, IN_F), lambda b, sel, bw: (b, 1, 0)),
            pl.BlockSpec((1, IN_F, JB), lambda b, sel, bw: (sel[b, 0], 0, 0)),
            pl.BlockSpec((1, IN_F, JB), lambda b, sel, bw: (sel[b, 0], 0, 1)),
            pl.BlockSpec((1, IN_F, JB), lambda b, sel, bw: (sel[b, 1], 0, 0)),
            pl.BlockSpec((1, IN_F, JB), lambda b, sel, bw: (sel[b, 1], 0, 1)),
            pl.BlockSpec((NUM_BANKS, OUT_F), lambda b, sel, bw: (0, 0)),
        ],
        out_specs=pl.BlockSpec((1, S, OUT_F), lambda b, sel, bw: (b, 0, 0)),
    )
    return pl.pallas_call(
        _body,
        grid_spec=grid_spec,
        out_shape=jax.ShapeDtypeStruct((B, S, OUT_F), jnp.float32),
    )(bank_selections, bank_weights, tensor, tensor, W, W, W, W, bias)


# speedup vs baseline: 1.0106x; 1.0068x over previous
import jax
import jax.numpy as jnp
from jax.experimental import pallas as pl
from jax.experimental.pallas import tpu as pltpu

B = 4
S = 2048
IN_F = 1024
OUT_F = 1024
NUM_BANKS = 16
SB = S // 2


def _body(sel_ref, bw_ref, xl_ref, xh_ref, w0_ref, w1_ref, bias_ref, out_ref):
    b = pl.program_id(0)
    bw0 = bw_ref[b, 0]
    bw1 = bw_ref[b, 1]
    w_eff = (bw0 * w0_ref[0] + bw1 * w1_ref[0]).astype(jnp.bfloat16)
    s0 = sel_ref[b, 0]
    s1 = sel_ref[b, 1]
    b_eff = (bw0 * bias_ref[s0, :] + bw1 * bias_ref[s1, :])[None, :]
    out_ref[0, :SB] = jnp.dot(xl_ref[0].astype(jnp.bfloat16), w_eff,
                              preferred_element_type=jnp.float32) + b_eff
    out_ref[0, SB:] = jnp.dot(xh_ref[0].astype(jnp.bfloat16), w_eff,
                              preferred_element_type=jnp.float32) + b_eff


def kernel(tensor, bank_weights, bank_selections, W, bias):
    grid_spec = pltpu.PrefetchScalarGridSpec(
        num_scalar_prefetch=2,
        grid=(B,),
        in_specs=[
            pl.BlockSpec((1, SB, IN_F), lambda b, sel, bw: (b, 0, 0)),
            pl.BlockSpec((1, SB, IN_F), lambda b, sel, bw: (b, 1, 0)),
            pl.BlockSpec((1, IN_F, OUT_F), lambda b, sel, bw: (sel[b, 0], 0, 0)),
            pl.BlockSpec((1, IN_F, OUT_F), lambda b, sel, bw: (sel[b, 1], 0, 0)),
            pl.BlockSpec((NUM_BANKS, OUT_F), lambda b, sel, bw: (0, 0)),
        ],
        out_specs=pl.BlockSpec((1, S, OUT_F), lambda b, sel, bw: (b, 0, 0)),
    )
    return pl.pallas_call(
        _body,
        grid_spec=grid_spec,
        out_shape=jax.ShapeDtypeStruct((B, S, OUT_F), jnp.float32),
    )(bank_selections, bank_weights, tensor, tensor, W, W, bias)
